# baseline (speedup 1.0000x reference)
import functools

import jax
import jax.numpy as jnp
from jax import lax
from jax.experimental import pallas as pl
from jax.experimental.pallas import tpu as pltpu
from jax.experimental.pallas import tpu_sc as plsc

B, T, L, V, D = 4096, 20, 100, 1000000, 64
LP = 112
NK = LP // 16
NW = 32
RPW = B // NW


def _emb_body(x_ref, tbl_ref, out_ref):
    x = x_ref[...]
    blk = x.shape[0]
    lane12 = lax.broadcasted_iota(jnp.int32, (blk, 12), 1)
    cnt = jnp.zeros((blk, 12), jnp.float32)
    for t in range(T):
        cnt = cnt + (x[:, t:t + 1] == lane12).astype(jnp.float32)
    acc = jnp.zeros((blk, D), jnp.float32)
    for v in range(12):
        acc = acc + cnt[:, v:v + 1] * tbl_ref[v:v + 1, :]
    out_ref[...] = acc


def _emb_tc(x, tbl):
    blk = 512
    return pl.pallas_call(
        _emb_body,
        grid=(B // blk,),
        in_specs=[
            pl.BlockSpec((blk, T), lambda i: (i, 0)),
            pl.BlockSpec((12, D), lambda i: (0, 0)),
        ],
        out_specs=pl.BlockSpec((blk, D), lambda i: (i, 0)),
        out_shape=jax.ShapeDtypeStruct((B, D), jnp.float32),
    )(x, tbl)


NS = 8


def _sc_body(map_hbm, lmh_hbm, lm_hbm, emb_hbm, out_hbm,
             lmh_v, lm_v, emb_v, rows_v, log_v, sems):
    c = lax.axis_index("c")
    s = lax.axis_index("s")
    wid = s * 2 + c
    base = wid * RPW

    pltpu.sync_copy(lmh_hbm.at[pl.ds(base * LP, RPW * LP)], lmh_v)
    pltpu.sync_copy(lm_hbm.at[pl.ds(base * LP, RPW * LP)], lm_v)
    pltpu.sync_copy(emb_hbm.at[pl.ds(base * D, RPW * D)], emb_v)

    iota = jnp.arange(16, dtype=jnp.int32)

    NBLK = RPW * NK

    def fire(j, buf, sem):
        idx = lmh_v[pl.ds(16 * j, 16)]
        pltpu.make_async_copy(map_hbm.at[idx], buf, sem).start()

    def drain(buf, sem):
        pltpu.make_async_copy(map_hbm.at[pl.ds(0, 16)], buf, sem).wait()

    def compute(j, buf):
        r = lax.div(j, NK)
        e = [emb_v[pl.ds(r * D + 16 * c, 16)] for c in range(4)]
        out = jnp.zeros((16,), jnp.float32)
        for i in range(16):
            lmi = plsc.load_gather(lm_v, [jnp.full((16,), 16 * j + i,
                                                   jnp.int32)])
            paroff = lax.shift_left(lax.bitwise_and(lmi, 1), 6)
            row = jnp.full((16,), i, jnp.int32)
            a = [plsc.load_gather(buf, [row, paroff + (iota + 16 * c)])
                 for c in range(4)]
            t = a[0] * e[0] + a[1] * e[1] + a[2] * e[2] + a[3] * e[3]
            out = jnp.where(iota == i, jnp.sum(t), out)
        log_v[pl.ds(16 * j, 16)] = out

    bufs = [rows_v[i] for i in range(NS)]
    for i in range(NS - 1):
        fire(jnp.int32(i), bufs[i], sems.at[i])

    def group_body(g, carry):
        j0 = NS * g
        for i in range(NS):
            nb = (i + NS - 1) % NS

            @pl.when(j0 + i + NS - 1 < NBLK)
            def _():
                fire(j0 + i + NS - 1, bufs[nb], sems.at[nb])

            drain(bufs[i], sems.at[i])
            compute(j0 + i, bufs[i])
        return carry

    lax.fori_loop(0, NBLK // NS, group_body, 0)

    pltpu.sync_copy(log_v, out_hbm.at[pl.ds(base * LP, RPW * LP)])


@functools.cache
def _sc_call():
    return pl.kernel(
        _sc_body,
        out_type=jax.ShapeDtypeStruct((B * LP,), jnp.float32),
        mesh=plsc.VectorSubcoreMesh(core_axis_name="c", subcore_axis_name="s"),
        scratch_types=[
            pltpu.VMEM((RPW * LP,), jnp.int32),
            pltpu.VMEM((RPW * LP,), jnp.int32),
            pltpu.VMEM((RPW * D,), jnp.float32),
            [pltpu.VMEM((16, 128), jnp.float32)
             for _ in range(NS)],
            pltpu.VMEM((RPW * LP,), jnp.float32),
            pltpu.SemaphoreType.DMA((NS,)),
        ],
        compiler_params=pltpu.CompilerParams(
            needs_layout_passes=False, use_tc_tiling_on_sc=True),
    )


def _loss_body(lg_ref, y_ref, g_ref, loss_ref, acc_ref):
    @pl.when(pl.program_id(0) == 0)
    def _():
        loss_ref[...] = jnp.zeros((1, 1), jnp.float32)
        acc_ref[...] = jnp.zeros((1, 1), jnp.float32)

    lg = lg_ref[...]
    blk = lg.shape[0]
    lane = lax.broadcasted_iota(jnp.int32, (blk, LP), 1)
    valid = lane < L
    neg = jnp.float32(-3.0e38)
    lgm = jnp.where(valid, lg, neg)
    m = jnp.max(lgm, axis=1, keepdims=True)
    ex = jnp.where(valid, jnp.exp(lg - m), 0.0)
    s = jnp.sum(ex, axis=1, keepdims=True)
    prob = ex / s
    pm = jnp.max(jnp.where(valid, prob, neg), axis=1, keepdims=True)
    pex = jnp.where(valid, jnp.exp(prob - pm), 0.0)
    logps = jnp.log(jnp.sum(pex, axis=1, keepdims=True))
    yv = y_ref[...]
    pick = jnp.sum(jnp.where(lane == yv, prob, 0.0), axis=1, keepdims=True)
    logp_y = pick - pm - logps
    loss_ref[...] += -jnp.sum(logp_y, axis=0, keepdims=True) / B
    z = jnp.where(valid, jnp.log(prob + 1e-20) + g_ref[...], neg)
    zm = jnp.max(z, axis=1, keepdims=True)
    pred = jnp.min(jnp.where(z == zm, lane, LP), axis=1, keepdims=True)
    acc_ref[...] += jnp.sum((pred == yv).astype(jnp.float32),
                            axis=0, keepdims=True) / B


def _loss_tc(logits, yv, g):
    blk = 512
    return pl.pallas_call(
        _loss_body,
        grid=(B // blk,),
        in_specs=[
            pl.BlockSpec((blk, LP), lambda i: (i, 0)),
            pl.BlockSpec((blk, 1), lambda i: (i, 0)),
            pl.BlockSpec((blk, LP), lambda i: (i, 0)),
        ],
        out_specs=[
            pl.BlockSpec((1, 1), lambda i: (0, 0)),
            pl.BlockSpec((1, 1), lambda i: (0, 0)),
        ],
        out_shape=[
            jax.ShapeDtypeStruct((1, 1), jnp.float32),
            jax.ShapeDtypeStruct((1, 1), jnp.float32),
        ],
    )(logits, yv, g)


def kernel(X_goldstandard, landmarks, y, goldstandard_table, emb_map_table):
    x = X_goldstandard.astype(jnp.int32)
    lm = landmarks.astype(jnp.int32)
    yv = y.astype(jnp.int32)
    emb = _emb_tc(x, goldstandard_table)
    lm_pad = jnp.concatenate(
        [lm, jnp.zeros((B, LP - L), jnp.int32)], axis=1)
    map_pairs = emb_map_table.reshape(V // 2, 2 * D)
    logits_flat = _sc_call()(
        map_pairs,
        lax.shift_right_logical(lm_pad, 1).reshape(-1),
        lm_pad.reshape(-1),
        emb.reshape(-1),
    )
    g = jax.random.gumbel(jax.random.key(1), (B, L), jnp.float32)
    g_pad = jnp.concatenate(
        [g, jnp.zeros((B, LP - L), jnp.float32)], axis=1)
    loss2, acc2 = _loss_tc(logits_flat.reshape(B, LP), yv, g_pad)
    return (loss2[0, 0], acc2[0, 0])


# speedup vs baseline: 1.6067x; 1.6067x over previous
import functools

import jax
import jax.numpy as jnp
from jax import lax
from jax.experimental import pallas as pl
from jax.experimental.pallas import tpu as pltpu
from jax.experimental.pallas import tpu_sc as plsc

B, T, L, V, D = 4096, 20, 100, 1000000, 64
LP = 112
NK = LP // 16
NW = 32
RPW = B // NW


def _emb_body(x_ref, tbl_ref, out_ref):
    x = x_ref[...]
    blk = x.shape[0]
    lane12 = lax.broadcasted_iota(jnp.int32, (blk, 12), 1)
    cnt = jnp.zeros((blk, 12), jnp.float32)
    for t in range(T):
        cnt = cnt + (x[:, t:t + 1] == lane12).astype(jnp.float32)
    acc = jnp.zeros((blk, D), jnp.float32)
    for v in range(12):
        acc = acc + cnt[:, v:v + 1] * tbl_ref[v:v + 1, :]
    out_ref[...] = acc


def _emb_tc(x, tbl):
    blk = 512
    return pl.pallas_call(
        _emb_body,
        grid=(B // blk,),
        in_specs=[
            pl.BlockSpec((blk, T), lambda i: (i, 0)),
            pl.BlockSpec((12, D), lambda i: (0, 0)),
        ],
        out_specs=pl.BlockSpec((blk, D), lambda i: (i, 0)),
        out_shape=jax.ShapeDtypeStruct((B, D), jnp.float32),
    )(x, tbl)


NS = 8


def _sc_body(map_hbm, lm_hbm, emb_hbm, out_hbm,
             lm_v, emb_v, rows_v, log_v, sems):
    c = lax.axis_index("c")
    s = lax.axis_index("s")
    wid = s * 2 + c
    base = wid * RPW

    pltpu.sync_copy(lm_hbm.at[pl.ds(base * LP, RPW * LP)], lm_v)
    pltpu.sync_copy(emb_hbm.at[pl.ds(base * D, RPW * D)], emb_v)

    iota = jnp.arange(16, dtype=jnp.int32)

    NBLK = RPW * NK

    def fire(j, buf, sem):
        idx = lm_v[pl.ds(16 * j, 16)]
        pltpu.make_async_copy(map_hbm.at[idx], buf, sem).start()

    def drain(buf, sem):
        pltpu.make_async_copy(map_hbm.at[pl.ds(0, 16)], buf, sem).wait()

    def compute(j, buf):
        r = lax.div(j, NK)
        e = [emb_v[pl.ds(r * D + 16 * c, 16)] for c in range(4)]
        out = jnp.zeros((16,), jnp.float32)
        for i in range(16):
            row = jnp.full((16,), i, jnp.int32)
            a = [plsc.load_gather(buf, [row, iota + 16 * c])
                 for c in range(4)]
            t = a[0] * e[0] + a[1] * e[1] + a[2] * e[2] + a[3] * e[3]
            out = jnp.where(iota == i, jnp.sum(t), out)
        log_v[pl.ds(16 * j, 16)] = out

    bufs = [rows_v[i] for i in range(NS)]
    for i in range(NS - 1):
        fire(jnp.int32(i), bufs[i], sems.at[i])

    def group_body(g, carry):
        j0 = NS * g
        for i in range(NS):
            nb = (i + NS - 1) % NS

            @pl.when(j0 + i + NS - 1 < NBLK)
            def _():
                fire(j0 + i + NS - 1, bufs[nb], sems.at[nb])

            drain(bufs[i], sems.at[i])
            compute(j0 + i, bufs[i])
        return carry

    lax.fori_loop(0, NBLK // NS, group_body, 0)

    pltpu.sync_copy(log_v, out_hbm.at[pl.ds(base * LP, RPW * LP)])


@functools.cache
def _sc_call():
    return pl.kernel(
        _sc_body,
        out_type=jax.ShapeDtypeStruct((B * LP,), jnp.float32),
        mesh=plsc.VectorSubcoreMesh(core_axis_name="c", subcore_axis_name="s"),
        scratch_types=[
            pltpu.VMEM((RPW * LP,), jnp.int32),
            pltpu.VMEM((RPW * D,), jnp.float32),
            [pltpu.VMEM((16, D), jnp.float32)
             for _ in range(NS)],
            pltpu.VMEM((RPW * LP,), jnp.float32),
            pltpu.SemaphoreType.DMA((NS,)),
        ],
        compiler_params=pltpu.CompilerParams(
            needs_layout_passes=False, use_tc_tiling_on_sc=False),
    )


def _loss_body(lg_ref, y_ref, g_ref, loss_ref, acc_ref):
    @pl.when(pl.program_id(0) == 0)
    def _():
        loss_ref[...] = jnp.zeros((1, 1), jnp.float32)
        acc_ref[...] = jnp.zeros((1, 1), jnp.float32)

    lg = lg_ref[...]
    blk = lg.shape[0]
    lane = lax.broadcasted_iota(jnp.int32, (blk, LP), 1)
    valid = lane < L
    neg = jnp.float32(-3.0e38)
    lgm = jnp.where(valid, lg, neg)
    m = jnp.max(lgm, axis=1, keepdims=True)
    ex = jnp.where(valid, jnp.exp(lg - m), 0.0)
    s = jnp.sum(ex, axis=1, keepdims=True)
    prob = ex / s
    pm = jnp.max(jnp.where(valid, prob, neg), axis=1, keepdims=True)
    pex = jnp.where(valid, jnp.exp(prob - pm), 0.0)
    logps = jnp.log(jnp.sum(pex, axis=1, keepdims=True))
    yv = y_ref[...]
    pick = jnp.sum(jnp.where(lane == yv, prob, 0.0), axis=1, keepdims=True)
    logp_y = pick - pm - logps
    loss_ref[...] += -jnp.sum(logp_y, axis=0, keepdims=True) / B
    z = jnp.where(valid, jnp.log(prob + 1e-20) + g_ref[...], neg)
    zm = jnp.max(z, axis=1, keepdims=True)
    pred = jnp.min(jnp.where(z == zm, lane, LP), axis=1, keepdims=True)
    acc_ref[...] += jnp.sum((pred == yv).astype(jnp.float32),
                            axis=0, keepdims=True) / B


def _loss_tc(logits, yv, g):
    blk = 512
    return pl.pallas_call(
        _loss_body,
        grid=(B // blk,),
        in_specs=[
            pl.BlockSpec((blk, LP), lambda i: (i, 0)),
            pl.BlockSpec((blk, 1), lambda i: (i, 0)),
            pl.BlockSpec((blk, LP), lambda i: (i, 0)),
        ],
        out_specs=[
            pl.BlockSpec((1, 1), lambda i: (0, 0)),
            pl.BlockSpec((1, 1), lambda i: (0, 0)),
        ],
        out_shape=[
            jax.ShapeDtypeStruct((1, 1), jnp.float32),
            jax.ShapeDtypeStruct((1, 1), jnp.float32),
        ],
    )(logits, yv, g)


def kernel(X_goldstandard, landmarks, y, goldstandard_table, emb_map_table):
    x = X_goldstandard.astype(jnp.int32)
    lm = landmarks.astype(jnp.int32)
    yv = y.astype(jnp.int32)
    emb = _emb_tc(x, goldstandard_table)
    lm_pad = jnp.concatenate(
        [lm, jnp.zeros((B, LP - L), jnp.int32)], axis=1)
    logits_flat = _sc_call()(
        emb_map_table,
        lm_pad.reshape(-1),
        emb.reshape(-1),
    )
    g = jax.random.gumbel(jax.random.key(1), (B, L), jnp.float32)
    g_pad = jnp.concatenate(
        [g, jnp.zeros((B, LP - L), jnp.float32)], axis=1)
    loss2, acc2 = _loss_tc(logits_flat.reshape(B, LP), yv, g_pad)
    return (loss2[0, 0], acc2[0, 0])
